# Initial kernel scaffold; baseline (speedup 1.0000x reference)
import jax
import jax.numpy as jnp
from jax.experimental import pallas as pl


def kernel(x, batch, gW0, gb0, gW1, gb1, gW2, gb2, gW3, gb3, gW4, gb4, tW, tb, sW0, sb0, bng0, bnb0, sW1, sb1, bng1, bnb1, sW2, sb2, bng2, bnb2, alpha, pW, pb):
    raise NotImplementedError("write your pallas kernel here")


# speedup vs baseline: 1.4954x; 1.4954x over previous
import functools

import jax
import jax.numpy as jnp
from jax import lax
from jax.experimental import pallas as pl
from jax.experimental.pallas import tpu as pltpu
from jax.experimental.pallas import tpu_sc as plsc

N = 100000
D = 128
F = 128
G = 1024
NC_OUT = 40
EPS = 1e-5

TILE = 2000
NWORK = 32
SEG_W = G // NWORK
RB = 512


def _pad128(a):
    out = jnp.zeros((F, F), jnp.float32)
    return out.at[: a.shape[0], : a.shape[1]].set(a)


def _node_mlp_kernel(x_ref, ws_ref, bs_ref, o_ref):
    x = x_ref[...]
    h = x
    for k in range(5):
        h = jnp.dot(h, ws_ref[k], preferred_element_type=jnp.float32)
        h = jnp.maximum(h + bs_ref[k : k + 1, :], 0.0)
    gout = jnp.dot(h, ws_ref[5], preferred_element_type=jnp.float32)
    gout = gout + bs_ref[5:6, :]
    s = x
    for k in range(6, 9):
        s = jnp.dot(s, ws_ref[k], preferred_element_type=jnp.float32)
        s = s + bs_ref[k : k + 1, :]
        if k < 8:
            s = jnp.maximum(s, 0.0)
    o_ref[...] = gout + s


def _pool_proj_kernel(p_ref, w_ref, b_ref, o_ref):
    o_ref[...] = (
        jnp.dot(p_ref[...], w_ref[...], preferred_element_type=jnp.float32)
        + b_ref[...]
    )


def _segmax_kernel(node_hbm, ids_hbm, offs_hbm, out_hbm, acc_v, rows_v, ids_v, offs_v):
    wid = lax.axis_index("s") * 2 + lax.axis_index("c")
    seg0 = wid * SEG_W
    pltpu.sync_copy(offs_hbm.at[pl.ds(seg0, 48)], offs_v)
    r0 = offs_v[pl.ds(0, 16)][0]
    r1 = offs_v[pl.ds(SEG_W, 16)][0]
    ra = (r0 // 8) * 8
    rz = ((r1 + 7) // 8) * 8
    neg = jnp.full((16,), -jnp.inf, jnp.float32)
    for si in range(SEG_W):
        for j in range(F // 16):
            acc_v[si, pl.ds(j * 16, 16)] = neg
    nblocks = (rz - ra + RB - 1) // RB

    def block_body(b, carry):
        rs = jnp.minimum(ra + b * RB, N - RB)
        pltpu.sync_copy(node_hbm.at[pl.ds(rs, RB)], rows_v)
        pltpu.sync_copy(ids_hbm.at[pl.ds(rs, RB)], ids_v)

        def group_body(gi, c2):
            idv = ids_v[pl.ds(gi * 16, 16)] - seg0
            for lane in range(16):
                s = idv[lane]

                @pl.when(jnp.logical_and(s >= 0, s < SEG_W))
                def _():
                    for j in range(F // 16):
                        sl = pl.ds(j * 16, 16)
                        acc_v[s, sl] = jnp.maximum(
                            acc_v[s, sl], rows_v[gi * 16 + lane, sl]
                        )

            return c2

        return lax.fori_loop(0, RB // 16, group_body, carry)

    lax.fori_loop(0, nblocks, block_body, 0)
    pltpu.sync_copy(acc_v, out_hbm.at[pl.ds(seg0, SEG_W)])


def kernel(x, batch, gW0, gb0, gW1, gb1, gW2, gb2, gW3, gb3, gW4, gb4,
           tW, tb, sW0, sb0, bng0, bnb0, sW1, sb1, bng1, bnb1,
           sW2, sb2, bng2, bnb2, alpha, pW, pb):
    f32 = jnp.float32
    one_m_a = (1.0 - alpha).astype(f32)
    a = alpha.astype(f32)
    inv = 1.0 / jnp.sqrt(1.0 + EPS)

    mats = []
    bias_rows = []
    for W, b in ((gW0, gb0), (gW1, gb1), (gW2, gb2), (gW3, gb3), (gW4, gb4)):
        mats.append(_pad128(W.T))
        bias_rows.append((b, 1.0))
    mats.append(_pad128(tW.T * one_m_a))
    bias_rows.append((tb, one_m_a))
    side = ((sW0, sb0, bng0, bnb0), (sW1, sb1, bng1, bnb1), (sW2, sb2, bng2, bnb2))
    for i, (W, b, gam, bet) in enumerate(side):
        scale = gam * inv
        gate = a if i == 2 else 1.0
        mats.append(_pad128(W.T * (scale * gate)[None, :]))
        bias_rows.append(((b * scale + bet), gate))
    ws = jnp.stack(mats)
    bs = jnp.zeros((16, F), f32)
    for k, (b, gate) in enumerate(bias_rows):
        bs = bs.at[k, : b.shape[0]].set(b * gate)

    node = pl.pallas_call(
        _node_mlp_kernel,
        grid=(N // TILE,),
        in_specs=[
            pl.BlockSpec((TILE, D), lambda i: (i, 0)),
            pl.BlockSpec((9, F, F), lambda i: (0, 0, 0)),
            pl.BlockSpec((16, F), lambda i: (0, 0)),
        ],
        out_specs=pl.BlockSpec((TILE, F), lambda i: (i, 0)),
        out_shape=jax.ShapeDtypeStruct((N, F), f32),
    )(x, ws, bs)

    ids = batch.astype(jnp.int32)
    offs = jnp.searchsorted(ids, jnp.arange(G, dtype=jnp.int32)).astype(jnp.int32)
    offs = jnp.concatenate([offs, jnp.full((16,), N, jnp.int32)])

    mesh = plsc.VectorSubcoreMesh(
        core_axis_name="c", subcore_axis_name="s", num_cores=2, num_subcores=16
    )
    pooled = pl.kernel(
        _segmax_kernel,
        out_type=jax.ShapeDtypeStruct((G, F), f32),
        mesh=mesh,
        scratch_types=[
            pltpu.VMEM((SEG_W, F), f32),
            pltpu.VMEM((RB, F), f32),
            pltpu.VMEM((RB,), jnp.int32),
            pltpu.VMEM((48,), jnp.int32),
        ],
    )(node, ids, offs)

    pwt = _pad128(pW.T)
    pbp = jnp.zeros((1, F), f32).at[0, : pb.shape[0]].set(pb)
    out = pl.pallas_call(
        _pool_proj_kernel,
        out_shape=jax.ShapeDtypeStruct((G, F), f32),
    )(pooled, pwt, pbp)
    return out[:, :NC_OUT]


# speedup vs baseline: 4.1236x; 2.7574x over previous
import jax
import jax.numpy as jnp
import numpy as np
from jax import lax
from jax.experimental import pallas as pl
from jax.experimental.pallas import tpu as pltpu
from jax.experimental.pallas import tpu_sc as plsc

N = 100000
D = 128
F = 128
G = 1024
NC_OUT = 40
EPS = 1e-5

TILE = 2000
NWORK = 32
SEG_W = G // NWORK
RB = 512
RB2 = 256


def _pad128(a):
    out = jnp.zeros((F, F), jnp.float32)
    return out.at[: a.shape[0], : a.shape[1]].set(a)


def _node_mlp_kernel(x_ref, ws_ref, bs_ref, o_ref):
    h = x_ref[...]
    for k in range(5):
        h = jnp.dot(h, ws_ref[k], preferred_element_type=jnp.float32)
        h = jnp.maximum(h + bs_ref[k : k + 1, :], 0.0)
    o_ref[...] = (
        jnp.dot(h, ws_ref[5], preferred_element_type=jnp.float32) + bs_ref[5:6, :]
    )


def _pool_proj_kernel(p_ref, w_ref, b_ref, o_ref):
    o_ref[...] = (
        jnp.dot(p_ref[...], w_ref[...], preferred_element_type=jnp.float32)
        + b_ref[...]
    )


def _segmax_kernel(node_hbm, ids_hbm, out_hbm, acc_v, rows_v, ids0_v, ids1_v,
                   probe_v, sem_a, sem_b):
    wid = lax.axis_index("s") * 2 + lax.axis_index("c")
    seg0 = wid * SEG_W

    def max_block_below(t):
        def sbody(_, lohi):
            lo, hi = lohi
            k = jnp.maximum((lo + hi) // 2, 0)
            pltpu.sync_copy(ids_hbm.at[pl.ds(k * 8, 16)], probe_v)
            lt = probe_v[...][0] < t
            return (jnp.where(lt, k, lo), jnp.where(lt, hi, k))

        lo, _ = lax.fori_loop(
            0, 14, sbody, (jnp.int32(-1), jnp.int32(N // 8))
        )
        return lo

    ra = jnp.maximum(max_block_below(seg0), 0) * 8
    rz = (max_block_below(seg0 + SEG_W) + 1) * 8
    nblocks = (rz - ra + RB2 - 1) // RB2
    npairs = (nblocks + 1) // 2

    neg = jnp.full((16,), -jnp.inf, jnp.float32)
    for si in range(SEG_W + 2):
        for j in range(F // 16):
            acc_v[si, pl.ds(j * 16, 16)] = neg

    ids_bufs = (ids0_v, ids1_v)

    def start_fetch(b, slot, sem):
        rs = jnp.minimum(ra + b * RB2, N + RB - RB2)
        pltpu.async_copy(node_hbm.at[pl.ds(rs, RB2)], rows_v.at[slot], sem)
        pltpu.async_copy(ids_hbm.at[pl.ds(rs, RB2)], ids_bufs[slot], sem)

    def wait_fetch(slot, sem):
        pltpu.make_async_copy(node_hbm.at[pl.ds(0, RB2)], rows_v.at[slot], sem).wait()
        pltpu.make_async_copy(ids_hbm.at[pl.ds(0, RB2)], ids_bufs[slot], sem).wait()

    def process_block(slot, carry):
        def group_body(gi, c2):
            sp = c2[0]
            regs = list(c2[1:])
            idv = jnp.clip(
                ids_bufs[slot][pl.ds(gi * 16, 16)] - seg0 + 1, 0, SEG_W + 1
            )
            for lane in range(16):
                s = idv[lane]
                pen = jnp.where(s != sp, -jnp.inf, 0.0).astype(jnp.float32)
                row = gi * 16 + lane
                for j in range(F // 16):
                    sl = pl.ds(j * 16, 16)
                    regs[j] = jnp.maximum(regs[j] + pen, rows_v[slot, row, sl])
                for j in range(F // 16):
                    acc_v[s, pl.ds(j * 16, 16)] = regs[j]
                sp = s
            return (sp, *regs)

        return lax.fori_loop(0, RB2 // 16, group_body, carry)

    start_fetch(0, 0, sem_a)
    start_fetch(1, 1, sem_b)

    def pair_body(p, carry):
        wait_fetch(0, sem_a)
        carry = process_block(0, carry)

        @pl.when(p + 1 < npairs)
        def _():
            start_fetch(p * 2 + 2, 0, sem_a)

        wait_fetch(1, sem_b)
        carry = process_block(1, carry)

        @pl.when(p + 1 < npairs)
        def _():
            start_fetch(p * 2 + 3, 1, sem_b)

        return carry

    init = (jnp.int32(-1),) + tuple(
        jnp.full((16,), -jnp.inf, jnp.float32) for _ in range(F // 16)
    )
    lax.fori_loop(0, npairs, pair_body, init)
    pltpu.sync_copy(acc_v.at[pl.ds(1, SEG_W)], out_hbm.at[pl.ds(seg0, SEG_W)])


def _prep_kernel(gW0, gW1, gW2, gW3, gW4, tW, sW0, sW1, sW2, pW,
                 gb0, gb1, gb2, gb3, gb4, tb, sb0, sb1, sb2,
                 bng0, bnb0, bng1, bnb1, bng2, bnb2, pb, alpha,
                 ws_ref, bs_ref, pwt_ref, pbp_ref):
    f32 = jnp.float32
    HID, SH = 100, 16
    inv = float(1.0 / np.sqrt(1.0 + EPS))
    a = alpha[0, 0]
    oma = 1.0 - a
    s0 = bng0[...] * inv
    s1 = bng1[...] * inv
    s2 = bng2[...] * inv

    def padc(m):
        return jnp.concatenate(
            [m, jnp.zeros((m.shape[0], F - m.shape[1]), f32)], axis=1
        )

    def padr(m):
        return jnp.concatenate(
            [m, jnp.zeros((F - m.shape[0], m.shape[1]), f32)], axis=0
        )

    def blkdiag(tl, br):
        top = jnp.concatenate([tl, jnp.zeros((HID, SH), f32)], axis=1)
        bot = jnp.concatenate([jnp.zeros((SH, HID), f32), br], axis=1)
        return padr(padc(jnp.concatenate([top, bot], axis=0)))

    ii = lax.broadcasted_iota(jnp.int32, (SH, SH), 0)
    jj = lax.broadcasted_iota(jnp.int32, (SH, SH), 1)
    eye = (ii == jj).astype(f32)

    ws_ref[0] = padc(jnp.concatenate([gW0[...].T, sW0[...].T * s0], axis=1))
    ws_ref[1] = blkdiag(gW1[...].T, sW1[...].T * s1)
    ws_ref[2] = blkdiag(gW2[...].T, eye)
    ws_ref[3] = blkdiag(gW3[...].T, eye)
    ws_ref[4] = blkdiag(gW4[...].T, eye)
    ws_ref[5] = padr(
        padc(jnp.concatenate([tW[...].T * oma, sW2[...].T * (s2 * a)], axis=0))
    )
    b0 = padc(
        jnp.concatenate([gb0[...], sb0[...] * s0 + bnb0[...]], axis=1)
    )
    b1 = padc(
        jnp.concatenate([gb1[...], sb1[...] * s1 + bnb1[...]], axis=1)
    )
    b5 = padc(tb[...] * oma + (sb2[...] * s2 + bnb2[...]) * a)
    bs_ref[...] = jnp.concatenate(
        [b0, b1, padc(gb2[...]), padc(gb3[...]), padc(gb4[...]), b5,
         jnp.zeros((2, F), f32)],
        axis=0,
    )
    pwt_ref[...] = padc(padr(pW[...].T))
    pbp_ref[...] = padc(pb[...])


def kernel(x, batch, gW0, gb0, gW1, gb1, gW2, gb2, gW3, gb3, gW4, gb4,
           tW, tb, sW0, sb0, bng0, bnb0, sW1, sb1, bng1, bnb1,
           sW2, sb2, bng2, bnb2, alpha, pW, pb):
    f32 = jnp.float32
    r1 = lambda v: v.reshape(1, -1)
    ws, bs, pwt, pbp = pl.pallas_call(
        _prep_kernel,
        out_shape=[
            jax.ShapeDtypeStruct((6, F, F), f32),
            jax.ShapeDtypeStruct((8, F), f32),
            jax.ShapeDtypeStruct((F, F), f32),
            jax.ShapeDtypeStruct((1, F), f32),
        ],
    )(gW0, gW1, gW2, gW3, gW4, tW, sW0, sW1, sW2, pW,
      r1(gb0), r1(gb1), r1(gb2), r1(gb3), r1(gb4), r1(tb),
      r1(sb0), r1(sb1), r1(sb2),
      r1(bng0), r1(bnb0), r1(bng1), r1(bnb1), r1(bng2), r1(bnb2),
      r1(pb), alpha.astype(f32).reshape(1, 1))

    node = pl.pallas_call(
        _node_mlp_kernel,
        grid=(N // TILE,),
        in_specs=[
            pl.BlockSpec((TILE, D), lambda i: (i, 0)),
            pl.BlockSpec((6, F, F), lambda i: (0, 0, 0)),
            pl.BlockSpec((8, F), lambda i: (0, 0)),
        ],
        out_specs=pl.BlockSpec((TILE, F), lambda i: (i, 0)),
        out_shape=jax.ShapeDtypeStruct((N + RB, F), f32),
    )(x, ws, bs)

    ids = jnp.concatenate(
        [batch.astype(jnp.int32), jnp.full((RB,), G, jnp.int32)]
    )

    mesh = plsc.VectorSubcoreMesh(
        core_axis_name="c", subcore_axis_name="s", num_cores=2, num_subcores=16
    )
    pooled = pl.kernel(
        _segmax_kernel,
        out_type=jax.ShapeDtypeStruct((G, F), f32),
        mesh=mesh,
        scratch_types=[
            pltpu.VMEM((SEG_W + 2, F), f32),
            pltpu.VMEM((2, RB2, F), f32),
            pltpu.VMEM((RB2,), jnp.int32),
            pltpu.VMEM((RB2,), jnp.int32),
            pltpu.VMEM((16,), jnp.int32),
            pltpu.SemaphoreType.DMA,
            pltpu.SemaphoreType.DMA,
        ],
    )(node, ids)

    out = pl.pallas_call(
        _pool_proj_kernel,
        out_shape=jax.ShapeDtypeStruct((G, F), f32),
    )(pooled, pwt, pbp)
    return out[:, :NC_OUT]
